# baseline (speedup 1.0000x reference)
import jax
import jax.numpy as jnp
from jax import lax
from jax.experimental import pallas as pl
from jax.experimental.pallas import tpu as pltpu
from jax.experimental.pallas import tpu_sc as plsc

N_FEATURES = 6
T = 0.001
B = 4096
L = 16
NC, NS = 2, 16
NW = NC * NS
BPW = B // NW
GROUPS = BPW // L
WORDS = 2 * 10
SFW = 2 * L * N_FEATURES
AUX = 256


def _iconst(v):
    return jnp.full((L,), v, jnp.int32)


def _sc_body(phi_hbm, aux_hbm, out_hbm, phi_v, aux_v, tab_v, out_v):
    wid = lax.axis_index("s") * NC + lax.axis_index("c")
    base = wid * BPW * WORDS

    pltpu.sync_copy(phi_hbm.at[pl.ds(base, BPW * WORDS)], phi_v)
    pltpu.sync_copy(aux_hbm, aux_v)

    iota = lax.iota(jnp.int32, L)
    wvec = [plsc.load_gather(aux_v, [_iconst(SFW + c)])
            for c in range(N_FEATURES)]

    cell6 = iota * 6
    v0 = jnp.zeros((L,), jnp.float32)
    v1 = jnp.zeros((L,), jnp.float32)
    for c in range(N_FEATURES):
        v0 = v0 + plsc.load_gather(aux_v, [cell6 + c]) * wvec[c]
        v1 = v1 + plsc.load_gather(aux_v, [cell6 + (96 + c)]) * wvec[c]
    m = jnp.maximum(v0, v1)
    e0 = jnp.exp((v0 - m) / T)
    e1 = jnp.exp((v1 - m) / T)
    tab_v[...] = (e0 * v0 + e1 * v1) / (e0 + e1)

    lane20 = iota * WORDS
    lane2 = iota * 2
    for g in range(GROUPS):
        gbase = g * L * WORDS
        delta = []
        for p in range(2):
            off = gbase + p * 10
            pr = jnp.zeros((L,), jnp.float32)
            for c in range(N_FEATURES):
                f = plsc.load_gather(phi_v, [lane20 + (off + c)])
                pr = pr + f.astype(jnp.float32) * wvec[c]
            x_ss = plsc.load_gather(phi_v, [lane20 + (off + 6)])
            y_ss = plsc.load_gather(phi_v, [lane20 + (off + 7)])
            x_es = plsc.load_gather(phi_v, [lane20 + (off + 8)])
            y_es = plsc.load_gather(phi_v, [lane20 + (off + 9)])
            v_ss = plsc.load_gather(tab_v, [x_ss * 4 + y_ss])
            v_es = plsc.load_gather(tab_v, [x_es * 4 + y_es])
            delta.append(pr + v_es - v_ss)
        d = delta[0] - delta[1]
        plsc.store_scatter(out_v, [lane2 + g * 2 * L],
                           1.0 / (1.0 + jnp.exp(-d)))
        plsc.store_scatter(out_v, [lane2 + (g * 2 * L + 1)],
                           1.0 / (1.0 + jnp.exp(d)))

    pltpu.sync_copy(out_v, out_hbm.at[pl.ds(wid * 2 * BPW, 2 * BPW)])


@jax.jit
def kernel(phi, succ_feats, W):
    phi_flat = phi.astype(jnp.int32).reshape(B * WORDS)
    aux = jnp.concatenate(
        [succ_feats.reshape(SFW), W.reshape(N_FEATURES),
         jnp.zeros((AUX - SFW - N_FEATURES,), jnp.float32)])

    mesh = plsc.VectorSubcoreMesh(core_axis_name="c", subcore_axis_name="s")
    run = pl.kernel(
        _sc_body,
        out_type=jax.ShapeDtypeStruct((2 * B,), jnp.float32),
        mesh=mesh,
        scratch_types=[
            pltpu.VMEM((BPW * WORDS,), jnp.int32),
            pltpu.VMEM((AUX,), jnp.float32),
            pltpu.VMEM((L,), jnp.float32),
            pltpu.VMEM((2 * BPW,), jnp.float32),
        ],
        compiler_params=pltpu.CompilerParams(needs_layout_passes=False),
    )
    return run(phi_flat, aux).reshape(B, 2, 1)


# speedup vs baseline: 1.3443x; 1.3443x over previous
import jax
import jax.numpy as jnp
from jax import lax
from jax.experimental import pallas as pl
from jax.experimental.pallas import tpu as pltpu
from jax.experimental.pallas import tpu_sc as plsc

N_FEATURES = 6
T = 0.001
B = 4096
L = 16
NC, NS = 2, 16
NW = NC * NS
BPW = B // NW
GROUPS = BPW // L
FIELDS = 10
SFW = 2 * L * N_FEATURES
AUX = 256
CHUNK = 2 * BPW


def _iconst(v):
    return jnp.full((L,), v, jnp.int32)


def _sc_body(phi_hbm, aux_hbm, out_hbm, phi_v, aux_v, tab_v, out_v):
    wid = lax.axis_index("s") * NC + lax.axis_index("c")

    for c in range(FIELDS):
        pltpu.sync_copy(phi_hbm.at[pl.ds(c * (2 * B) + wid * CHUNK, CHUNK)],
                        phi_v.at[pl.ds(c * CHUNK, CHUNK)])
    pltpu.sync_copy(aux_hbm, aux_v)

    iota = lax.iota(jnp.int32, L)
    wvec = [plsc.load_gather(aux_v, [_iconst(SFW + c)])
            for c in range(N_FEATURES)]

    cell6 = iota * 6
    v0 = jnp.zeros((L,), jnp.float32)
    v1 = jnp.zeros((L,), jnp.float32)
    for c in range(N_FEATURES):
        v0 = v0 + plsc.load_gather(aux_v, [cell6 + c]) * wvec[c]
        v1 = v1 + plsc.load_gather(aux_v, [cell6 + (96 + c)]) * wvec[c]
    m = jnp.maximum(v0, v1)
    e0 = jnp.exp((v0 - m) / T)
    e1 = jnp.exp((v1 - m) / T)
    tab_v[...] = (e0 * v0 + e1 * v1) / (e0 + e1)

    for g in range(GROUPS):
        delta = []
        for p in range(2):
            off = p * BPW + g * L
            pr = jnp.zeros((L,), jnp.float32)
            for c in range(N_FEATURES):
                f = phi_v[pl.ds(c * CHUNK + off, L)]
                pr = pr + f.astype(jnp.float32) * wvec[c]
            x_ss = phi_v[pl.ds(6 * CHUNK + off, L)]
            y_ss = phi_v[pl.ds(7 * CHUNK + off, L)]
            x_es = phi_v[pl.ds(8 * CHUNK + off, L)]
            y_es = phi_v[pl.ds(9 * CHUNK + off, L)]
            v_ss = plsc.load_gather(tab_v, [x_ss * 4 + y_ss])
            v_es = plsc.load_gather(tab_v, [x_es * 4 + y_es])
            delta.append(pr + v_es - v_ss)
        d = delta[0] - delta[1]
        out_v[pl.ds(g * L, L)] = 1.0 / (1.0 + jnp.exp(-d))
        out_v[pl.ds(BPW + g * L, L)] = 1.0 / (1.0 + jnp.exp(d))

    base = wid * BPW
    pltpu.sync_copy(out_v.at[pl.ds(0, BPW)], out_hbm.at[pl.ds(base, BPW)])
    pltpu.sync_copy(out_v.at[pl.ds(BPW, BPW)],
                    out_hbm.at[pl.ds(B + base, BPW)])


@jax.jit
def kernel(phi, succ_feats, W):
    phi_phys = (phi.astype(jnp.int32)
                .transpose(2, 1, 0)
                .reshape(FIELDS, 2, NW, BPW)
                .transpose(0, 2, 1, 3)
                .reshape(FIELDS * 2 * B))
    aux = jnp.concatenate(
        [succ_feats.reshape(SFW), W.reshape(N_FEATURES),
         jnp.zeros((AUX - SFW - N_FEATURES,), jnp.float32)])

    mesh = plsc.VectorSubcoreMesh(core_axis_name="c", subcore_axis_name="s")
    run = pl.kernel(
        _sc_body,
        out_type=jax.ShapeDtypeStruct((2 * B,), jnp.float32),
        mesh=mesh,
        scratch_types=[
            pltpu.VMEM((FIELDS * CHUNK,), jnp.int32),
            pltpu.VMEM((AUX,), jnp.float32),
            pltpu.VMEM((L,), jnp.float32),
            pltpu.VMEM((CHUNK,), jnp.float32),
        ],
        compiler_params=pltpu.CompilerParams(needs_layout_passes=False),
    )
    out_flat = run(phi_phys, aux)
    return out_flat.reshape(2, B).transpose(1, 0)[:, :, None]


# speedup vs baseline: 1.7051x; 1.2684x over previous
import jax
import jax.numpy as jnp
from jax import lax
from jax.experimental import pallas as pl
from jax.experimental.pallas import tpu as pltpu
from jax.experimental.pallas import tpu_sc as plsc

N_FEATURES = 6
T = 0.001
B = 4096
L = 16
NC, NS = 2, 16
NW = NC * NS
BPW = B // NW
GROUPS = BPW // L
FIELDS = 10
SFW = 2 * L * N_FEATURES
AUX = 256
CHUNK = 2 * BPW


def _iconst(v):
    return jnp.full((L,), v, jnp.int32)


def _sc_body(phi_hbm, aux_hbm, out_hbm, phi_v, aux_v, tab_v, out_v, dma_sem):
    wid = lax.axis_index("s") * NC + lax.axis_index("c")

    copies = [
        pltpu.async_copy(
            phi_hbm.at[pl.ds(c * (2 * B) + wid * CHUNK, CHUNK)],
            phi_v.at[pl.ds(c * CHUNK, CHUNK)],
            dma_sem,
        )
        for c in range(FIELDS)
    ]
    pltpu.sync_copy(aux_hbm, aux_v)
    for cp in copies:
        cp.wait()

    iota = lax.iota(jnp.int32, L)
    wvec = [plsc.load_gather(aux_v, [_iconst(SFW + c)])
            for c in range(N_FEATURES)]

    cell6 = iota * 6
    v0 = jnp.zeros((L,), jnp.float32)
    v1 = jnp.zeros((L,), jnp.float32)
    for c in range(N_FEATURES):
        v0 = v0 + plsc.load_gather(aux_v, [cell6 + c]) * wvec[c]
        v1 = v1 + plsc.load_gather(aux_v, [cell6 + (96 + c)]) * wvec[c]
    m = jnp.maximum(v0, v1)
    e0 = jnp.exp((v0 - m) / T)
    e1 = jnp.exp((v1 - m) / T)
    tab_v[...] = (e0 * v0 + e1 * v1) / (e0 + e1)

    for g in range(GROUPS):
        delta = []
        for p in range(2):
            off = p * BPW + g * L
            pr = jnp.zeros((L,), jnp.float32)
            for c in range(N_FEATURES):
                f = phi_v[pl.ds(c * CHUNK + off, L)]
                pr = pr + f.astype(jnp.float32) * wvec[c]
            x_ss = phi_v[pl.ds(6 * CHUNK + off, L)]
            y_ss = phi_v[pl.ds(7 * CHUNK + off, L)]
            x_es = phi_v[pl.ds(8 * CHUNK + off, L)]
            y_es = phi_v[pl.ds(9 * CHUNK + off, L)]
            v_ss = plsc.load_gather(tab_v, [x_ss * 4 + y_ss])
            v_es = plsc.load_gather(tab_v, [x_es * 4 + y_es])
            delta.append(pr + v_es - v_ss)
        d = delta[0] - delta[1]
        out_v[pl.ds(g * L, L)] = 1.0 / (1.0 + jnp.exp(-d))
        out_v[pl.ds(BPW + g * L, L)] = 1.0 / (1.0 + jnp.exp(d))

    base = wid * BPW
    pltpu.sync_copy(out_v.at[pl.ds(0, BPW)], out_hbm.at[pl.ds(base, BPW)])
    pltpu.sync_copy(out_v.at[pl.ds(BPW, BPW)],
                    out_hbm.at[pl.ds(B + base, BPW)])


@jax.jit
def kernel(phi, succ_feats, W):
    phi_phys = (phi.astype(jnp.int32)
                .transpose(2, 1, 0)
                .reshape(FIELDS, 2, NW, BPW)
                .transpose(0, 2, 1, 3)
                .reshape(FIELDS * 2 * B))
    aux = jnp.concatenate(
        [succ_feats.reshape(SFW), W.reshape(N_FEATURES),
         jnp.zeros((AUX - SFW - N_FEATURES,), jnp.float32)])

    mesh = plsc.VectorSubcoreMesh(core_axis_name="c", subcore_axis_name="s")
    run = pl.kernel(
        _sc_body,
        out_type=jax.ShapeDtypeStruct((2 * B,), jnp.float32),
        mesh=mesh,
        scratch_types=[
            pltpu.VMEM((FIELDS * CHUNK,), jnp.int32),
            pltpu.VMEM((AUX,), jnp.float32),
            pltpu.VMEM((L,), jnp.float32),
            pltpu.VMEM((CHUNK,), jnp.float32),
            pltpu.SemaphoreType.DMA,
        ],
        compiler_params=pltpu.CompilerParams(needs_layout_passes=False),
    )
    out_flat = run(phi_phys, aux)
    return out_flat.reshape(2, B, 1).transpose(1, 0, 2)


# speedup vs baseline: 1.7068x; 1.0010x over previous
import jax
import jax.numpy as jnp
from jax import lax
from jax.experimental import pallas as pl
from jax.experimental.pallas import tpu as pltpu
from jax.experimental.pallas import tpu_sc as plsc

N_FEATURES = 6
T = 0.001
B = 4096
L = 16
NC, NS = 2, 16
NW = NC * NS
BPW = B // NW
GROUPS = BPW // L
FIELDS = 10
SFW = 2 * L * N_FEATURES
AUX = 256
CHUNK = 2 * BPW


def _iconst(v):
    return jnp.full((L,), v, jnp.int32)


def _sc_body(phi_hbm, aux_hbm, out_hbm, phi_v, aux_v, tab_v, out_v, dma_sem):
    wid = lax.axis_index("s") * NC + lax.axis_index("c")

    copies = [
        pltpu.async_copy(
            phi_hbm.at[pl.ds(c * (2 * B) + wid * CHUNK, CHUNK)],
            phi_v.at[pl.ds(c * CHUNK, CHUNK)],
            dma_sem,
        )
        for c in range(FIELDS)
    ]
    pltpu.sync_copy(aux_hbm, aux_v)
    for cp in copies:
        cp.wait()

    iota = lax.iota(jnp.int32, L)
    wvec = [plsc.load_gather(aux_v, [_iconst(SFW + c)])
            for c in range(N_FEATURES)]

    cell6 = iota * 6
    v0 = jnp.zeros((L,), jnp.float32)
    v1 = jnp.zeros((L,), jnp.float32)
    for c in range(N_FEATURES):
        v0 = v0 + plsc.load_gather(aux_v, [cell6 + c]) * wvec[c]
        v1 = v1 + plsc.load_gather(aux_v, [cell6 + (96 + c)]) * wvec[c]
    m = jnp.maximum(v0, v1)
    e0 = jnp.exp((v0 - m) / T)
    e1 = jnp.exp((v1 - m) / T)
    tab_v[...] = (e0 * v0 + e1 * v1) / (e0 + e1)

    for g in range(GROUPS):
        goff = g * L
        delta = []
        for p in range(2):
            off = p * BPW + goff
            pr = jnp.zeros((L,), jnp.float32)
            for c in range(N_FEATURES):
                f = phi_v[pl.ds(c * CHUNK + off, L)]
                pr = pr + f.astype(jnp.float32) * wvec[c]
            x_ss = phi_v[pl.ds(6 * CHUNK + off, L)]
            y_ss = phi_v[pl.ds(7 * CHUNK + off, L)]
            x_es = phi_v[pl.ds(8 * CHUNK + off, L)]
            y_es = phi_v[pl.ds(9 * CHUNK + off, L)]
            v_ss = plsc.load_gather(tab_v, [x_ss * 4 + y_ss])
            v_es = plsc.load_gather(tab_v, [x_es * 4 + y_es])
            delta.append(pr + v_es - v_ss)
        d = delta[0] - delta[1]
        out_v[pl.ds(goff, L)] = 1.0 / (1.0 + jnp.exp(-d))
        out_v[pl.ds(BPW + goff, L)] = 1.0 / (1.0 + jnp.exp(d))

    base = wid * BPW
    pltpu.sync_copy(out_v.at[pl.ds(0, BPW)], out_hbm.at[pl.ds(base, BPW)])
    pltpu.sync_copy(out_v.at[pl.ds(BPW, BPW)],
                    out_hbm.at[pl.ds(B + base, BPW)])


@jax.jit
def kernel(phi, succ_feats, W):
    phi_phys = (phi.astype(jnp.int32)
                .transpose(2, 1, 0)
                .reshape(FIELDS, 2, NW, BPW)
                .transpose(0, 2, 1, 3)
                .reshape(FIELDS * 2 * B))
    aux = jnp.concatenate(
        [succ_feats.reshape(SFW), W.reshape(N_FEATURES),
         jnp.zeros((AUX - SFW - N_FEATURES,), jnp.float32)])

    mesh = plsc.VectorSubcoreMesh(core_axis_name="c", subcore_axis_name="s")
    run = pl.kernel(
        _sc_body,
        out_type=jax.ShapeDtypeStruct((2 * B,), jnp.float32),
        mesh=mesh,
        scratch_types=[
            pltpu.VMEM((FIELDS * CHUNK,), jnp.int32),
            pltpu.VMEM((AUX,), jnp.float32),
            pltpu.VMEM((L,), jnp.float32),
            pltpu.VMEM((CHUNK,), jnp.float32),
            pltpu.SemaphoreType.DMA,
        ],
        compiler_params=pltpu.CompilerParams(needs_layout_passes=False),
    )
    out_flat = run(phi_phys, aux)
    return out_flat.reshape(2, B, 1).transpose(1, 0, 2)


# speedup vs baseline: 1.7391x; 1.0190x over previous
import jax
import jax.numpy as jnp
from jax import lax
from jax.experimental import pallas as pl
from jax.experimental.pallas import tpu as pltpu
from jax.experimental.pallas import tpu_sc as plsc

N_FEATURES = 6
T = 0.001
B = 4096
L = 16
NC, NS = 2, 16
NW = NC * NS
BPW = B // NW
GROUPS = BPW // L
FIELDS = 10
SFW = 2 * L * N_FEATURES
AUX = 256
CHUNK = 2 * BPW


def _iconst(v):
    return jnp.full((L,), v, jnp.int32)


def _sc_body(phi_hbm, aux_hbm, out_hbm, phi_v, aux_v, tab_v, out_v, dma_sem):
    wid = lax.axis_index("s") * NC + lax.axis_index("c")

    copies = [
        pltpu.async_copy(
            phi_hbm.at[pl.ds(c * (2 * B) + wid * CHUNK, CHUNK)],
            phi_v.at[pl.ds(c * CHUNK, CHUNK)],
            dma_sem,
        )
        for c in range(FIELDS)
    ]
    pltpu.sync_copy(aux_hbm, aux_v)
    for cp in copies:
        cp.wait()

    iota = lax.iota(jnp.int32, L)
    wvec = [plsc.load_gather(aux_v, [_iconst(SFW + c)])
            for c in range(N_FEATURES)]

    cell6 = iota * 6
    v0 = jnp.zeros((L,), jnp.float32)
    v1 = jnp.zeros((L,), jnp.float32)
    for c in range(N_FEATURES):
        v0 = v0 + plsc.load_gather(aux_v, [cell6 + c]) * wvec[c]
        v1 = v1 + plsc.load_gather(aux_v, [cell6 + (96 + c)]) * wvec[c]
    m = jnp.maximum(v0, v1)
    e0 = jnp.exp((v0 - m) / T)
    e1 = jnp.exp((v1 - m) / T)
    tab_v[...] = (e0 * v0 + e1 * v1) / (e0 + e1)

    def group(g, carry):
        goff = pl.multiple_of(g * L, L)
        delta = []
        for p in range(2):
            off = pl.multiple_of(p * BPW + goff, L)
            pr = jnp.zeros((L,), jnp.float32)
            for c in range(N_FEATURES):
                f = phi_v[pl.ds(c * CHUNK + off, L)]
                pr = pr + f.astype(jnp.float32) * wvec[c]
            x_ss = phi_v[pl.ds(6 * CHUNK + off, L)]
            y_ss = phi_v[pl.ds(7 * CHUNK + off, L)]
            x_es = phi_v[pl.ds(8 * CHUNK + off, L)]
            y_es = phi_v[pl.ds(9 * CHUNK + off, L)]
            v_ss = plsc.load_gather(tab_v, [x_ss * 4 + y_ss])
            v_es = plsc.load_gather(tab_v, [x_es * 4 + y_es])
            delta.append(pr + v_es - v_ss)
        d = delta[0] - delta[1]
        out_v[pl.ds(goff, L)] = 1.0 / (1.0 + jnp.exp(-d))
        out_v[pl.ds(BPW + goff, L)] = 1.0 / (1.0 + jnp.exp(d))
        return carry

    lax.fori_loop(0, GROUPS, group, 0)

    base = wid * BPW
    pltpu.sync_copy(out_v.at[pl.ds(0, BPW)], out_hbm.at[pl.ds(base, BPW)])
    pltpu.sync_copy(out_v.at[pl.ds(BPW, BPW)],
                    out_hbm.at[pl.ds(B + base, BPW)])


@jax.jit
def kernel(phi, succ_feats, W):
    phi_phys = (phi.astype(jnp.int32)
                .transpose(2, 1, 0)
                .reshape(FIELDS, 2, NW, BPW)
                .transpose(0, 2, 1, 3)
                .reshape(FIELDS * 2 * B))
    aux = jnp.concatenate(
        [succ_feats.reshape(SFW), W.reshape(N_FEATURES),
         jnp.zeros((AUX - SFW - N_FEATURES,), jnp.float32)])

    mesh = plsc.VectorSubcoreMesh(core_axis_name="c", subcore_axis_name="s")
    run = pl.kernel(
        _sc_body,
        out_type=jax.ShapeDtypeStruct((2 * B,), jnp.float32),
        mesh=mesh,
        scratch_types=[
            pltpu.VMEM((FIELDS * CHUNK,), jnp.int32),
            pltpu.VMEM((AUX,), jnp.float32),
            pltpu.VMEM((L,), jnp.float32),
            pltpu.VMEM((CHUNK,), jnp.float32),
            pltpu.SemaphoreType.DMA,
        ],
        compiler_params=pltpu.CompilerParams(needs_layout_passes=False),
    )
    out_flat = run(phi_phys, aux)
    return out_flat.reshape(2, B, 1).transpose(1, 0, 2)
